# baseline (speedup 1.0000x reference)
import jax
import jax.numpy as jnp
from jax import lax
from jax.experimental import pallas as pl
from jax.experimental.pallas import tpu as pltpu

EPS = 1e-6
ROPE_BASE = 1e6
NUM_H = 16
NUM_HKV = 16
HEAD_DIM = 128


def _bdiv(n, pref):
    return pref if n % pref == 0 else n


def _ln_matmul_kernel(h_ref, w_ref, wm_ref, b_ref, o_ref):
    h = h_ref[...]
    v = jnp.mean(h * h, axis=-1, keepdims=True)
    x = h * lax.rsqrt(v + EPS) * w_ref[...]
    o_ref[...] = (
        jnp.dot(x, wm_ref[...], preferred_element_type=jnp.float32) + b_ref[...]
    )


def _qkv_proj(hidden, w_ln, WmT, b, bt, bn):
    T, D = hidden.shape
    N = WmT.shape[1]
    return pl.pallas_call(
        _ln_matmul_kernel,
        grid=(N // bn, T // bt),
        in_specs=[
            pl.BlockSpec((bt, D), lambda j, i: (i, 0)),
            pl.BlockSpec((1, D), lambda j, i: (0, 0)),
            pl.BlockSpec((D, bn), lambda j, i: (0, j)),
            pl.BlockSpec((1, bn), lambda j, i: (0, j)),
        ],
        out_specs=pl.BlockSpec((bt, bn), lambda j, i: (i, j)),
        out_shape=jax.ShapeDtypeStruct((T, N), jnp.float32),
    )(hidden, w_ln.reshape(1, D), WmT, b.reshape(1, N))


def _rope_kernel(x_ref, cos_ref, sin_ref, o_ref):
    x = x_ref[0]
    hd = x.shape[-1]
    x1 = x[:, : hd // 2]
    x2 = x[:, hd // 2:]
    c = cos_ref[...]
    s = sin_ref[...]
    o_ref[0] = jnp.concatenate([x1 * c - x2 * s, x2 * c + x1 * s], axis=-1)


def _rope(qk, cosv, sinv):
    nh, T, hd = qk.shape
    return pl.pallas_call(
        _rope_kernel,
        grid=(nh,),
        in_specs=[
            pl.BlockSpec((1, T, hd), lambda h: (h, 0, 0)),
            pl.BlockSpec((T, hd // 2), lambda h: (0, 0)),
            pl.BlockSpec((T, hd // 2), lambda h: (0, 0)),
        ],
        out_specs=pl.BlockSpec((1, T, hd), lambda h: (h, 0, 0)),
        out_shape=jax.ShapeDtypeStruct((nh, T, hd), jnp.float32),
    )(qk, cosv, sinv)


def _attn_kernel(q_ref, k_ref, v_ref, o_ref, *, scale):
    i = pl.program_id(1)
    q = q_ref[0]
    k = k_ref[0]
    s = lax.dot_general(
        q, k, (((1,), (1,)), ((), ())), preferred_element_type=jnp.float32
    ) * scale
    tq, tk = s.shape
    rows = i * tq + lax.broadcasted_iota(jnp.int32, (tq, tk), 0)
    cols = lax.broadcasted_iota(jnp.int32, (tq, tk), 1)
    s = jnp.where(cols <= rows, s, -1e30)
    m = jnp.max(s, axis=-1, keepdims=True)
    p = jnp.exp(s - m)
    l = jnp.sum(p, axis=-1, keepdims=True)
    o = lax.dot_general(
        p.astype(jnp.bfloat16), v_ref[0], (((1,), (0,)), ((), ())),
        preferred_element_type=jnp.float32,
    )
    o_ref[0] = o / l


def _attention(q, k, v, bq):
    import functools
    nh, T, hd = q.shape
    return pl.pallas_call(
        functools.partial(_attn_kernel, scale=hd ** -0.5),
        grid=(nh, T // bq),
        in_specs=[
            pl.BlockSpec((1, bq, hd), lambda h, i: (h, i, 0)),
            pl.BlockSpec((1, T, hd), lambda h, i: (h, 0, 0)),
            pl.BlockSpec((1, T, hd), lambda h, i: (h, 0, 0)),
        ],
        out_specs=pl.BlockSpec((1, bq, hd), lambda h, i: (h, i, 0)),
        out_shape=jax.ShapeDtypeStruct((nh, T, hd), jnp.float32),
    )(q, k, v)


def _oproj_kernel(o_ref_in, w_ref, hid_ref, out_ref):
    out_ref[...] = hid_ref[...] + jnp.dot(
        o_ref_in[...], w_ref[...], preferred_element_type=jnp.float32
    )


def _oproj(o_flat, WoT, hidden, bt, bn):
    T, N = o_flat.shape
    D = WoT.shape[1]
    return pl.pallas_call(
        _oproj_kernel,
        grid=(D // bn, T // bt),
        in_specs=[
            pl.BlockSpec((bt, N), lambda j, i: (i, 0)),
            pl.BlockSpec((N, bn), lambda j, i: (0, j)),
            pl.BlockSpec((bt, bn), lambda j, i: (i, j)),
        ],
        out_specs=pl.BlockSpec((bt, bn), lambda j, i: (i, j)),
        out_shape=jax.ShapeDtypeStruct((T, D), jnp.float32),
    )(o_flat, WoT, hidden)


def _router_kernel(h_ref, w_ref, wr_ref, x2_ref, tw_ref, ti_ref, sg_ref, *, n_e):
    h = h_ref[...]
    v = jnp.mean(h * h, axis=-1, keepdims=True)
    x2 = h * lax.rsqrt(v + EPS) * w_ref[...]
    x2_ref[...] = x2.astype(jnp.bfloat16)
    logits = jnp.dot(x2, wr_ref[...], preferred_element_type=jnp.float32)
    cols = lax.broadcasted_iota(jnp.int32, logits.shape, 1)
    rl = jnp.where(cols < n_e, logits, -1e30)
    m = jnp.max(rl, axis=-1, keepdims=True)
    p = jnp.exp(rl - m)
    rw = p / jnp.sum(p, axis=-1, keepdims=True)
    big = jnp.int32(10 ** 6)
    w1 = jnp.max(rw, axis=-1, keepdims=True)
    i1 = jnp.min(jnp.where(rw == w1, cols, big), axis=-1, keepdims=True)
    rw2 = jnp.where(cols == i1, -1.0, rw)
    w2 = jnp.max(rw2, axis=-1, keepdims=True)
    i2 = jnp.min(jnp.where(rw2 == w2, cols, big), axis=-1, keepdims=True)
    tw_ref[...] = jnp.concatenate([w1, w2], axis=1)
    ti_ref[...] = jnp.concatenate([i1, i2], axis=1)
    sg_ref[...] = 1.0 / (1.0 + jnp.exp(-logits[:, n_e:n_e + 1]))


def _router(h, w_ln2, wr, bt, n_e):
    import functools
    T, D = h.shape
    outs = pl.pallas_call(
        functools.partial(_router_kernel, n_e=n_e),
        grid=(T // bt,),
        in_specs=[
            pl.BlockSpec((bt, D), lambda i: (i, 0)),
            pl.BlockSpec((1, D), lambda i: (0, 0)),
            pl.BlockSpec((D, 128), lambda i: (0, 0)),
        ],
        out_specs=[
            pl.BlockSpec((bt, D), lambda i: (i, 0)),
            pl.BlockSpec((bt, 2), lambda i: (i, 0)),
            pl.BlockSpec((bt, 2), lambda i: (i, 0)),
            pl.BlockSpec((bt, 1), lambda i: (i, 0)),
        ],
        out_shape=[
            jax.ShapeDtypeStruct((T, D), jnp.bfloat16),
            jax.ShapeDtypeStruct((T, 2), jnp.float32),
            jax.ShapeDtypeStruct((T, 2), jnp.int32),
            jax.ShapeDtypeStruct((T, 1), jnp.float32),
        ],
    )(h, w_ln2.reshape(1, D), wr)
    return outs


def _gu_kernel(x_ref, wg_ref, wu_ref, o_ref):
    x = x_ref[...]
    g = jnp.dot(x, wg_ref[...], preferred_element_type=jnp.float32)
    u = jnp.dot(x, wu_ref[...], preferred_element_type=jnp.float32)
    act = g * (1.0 / (1.0 + jnp.exp(-g))) * u
    o_ref[...] = act.astype(jnp.bfloat16)


def _gu(x2, WgT, WuT, bt, bi):
    T, D = x2.shape
    I = WgT.shape[1]
    return pl.pallas_call(
        _gu_kernel,
        grid=(I // bi, T // bt),
        in_specs=[
            pl.BlockSpec((bt, D), lambda j, i: (i, 0)),
            pl.BlockSpec((D, bi), lambda j, i: (0, j)),
            pl.BlockSpec((D, bi), lambda j, i: (0, j)),
        ],
        out_specs=pl.BlockSpec((bt, bi), lambda j, i: (i, j)),
        out_shape=jax.ShapeDtypeStruct((T, I), jnp.bfloat16),
    )(x2, WgT, WuT)


def _down_kernel(a_ref, wd_ref, h_ref, sg_ref, o_ref):
    y = jnp.dot(a_ref[...], wd_ref[...], preferred_element_type=jnp.float32)
    o_ref[...] = h_ref[...] + y * sg_ref[...]


def _down(act, WdT, h, sg, bt, bn):
    T, I = act.shape
    D = WdT.shape[1]
    return pl.pallas_call(
        _down_kernel,
        grid=(D // bn, T // bt),
        in_specs=[
            pl.BlockSpec((bt, I), lambda j, i: (i, 0)),
            pl.BlockSpec((I, bn), lambda j, i: (0, j)),
            pl.BlockSpec((bt, bn), lambda j, i: (i, j)),
            pl.BlockSpec((bt, 1), lambda j, i: (i, 0)),
        ],
        out_specs=pl.BlockSpec((bt, bn), lambda j, i: (i, j)),
        out_shape=jax.ShapeDtypeStruct((T, D), jnp.float32),
    )(act, WdT, h, sg)


def _moe_kernel(x_ref, tw_ref, ti_ref, wgu_ref, wd_ref, part_ref, o_ref, *, i_moe):
    e = pl.program_id(1)
    x = x_ref[...]
    gu = lax.dot_general(
        x, wgu_ref[0], (((1,), (1,)), ((), ())), preferred_element_type=jnp.float32
    )
    g = gu[:, :i_moe]
    u = gu[:, i_moe:]
    act = (g * (1.0 / (1.0 + jnp.exp(-g))) * u).astype(jnp.bfloat16)
    y = jnp.dot(act, wd_ref[0], preferred_element_type=jnp.float32)
    sel = (ti_ref[...] == e).astype(jnp.float32)
    w_e = jnp.sum(tw_ref[...] * sel, axis=-1, keepdims=True)
    contrib = y * w_e

    @pl.when(e == 0)
    def _():
        o_ref[...] = part_ref[...] + contrib

    @pl.when(e != 0)
    def _():
        o_ref[...] = o_ref[...] + contrib


def _moe(x2, tw, ti, Wgu, WdT, part, bt):
    import functools
    T, D = x2.shape
    E, two_im, _ = Wgu.shape
    i_moe = two_im // 2
    return pl.pallas_call(
        functools.partial(_moe_kernel, i_moe=i_moe),
        grid=(T // bt, E),
        in_specs=[
            pl.BlockSpec((bt, D), lambda i, e: (i, 0)),
            pl.BlockSpec((bt, 2), lambda i, e: (i, 0)),
            pl.BlockSpec((bt, 2), lambda i, e: (i, 0)),
            pl.BlockSpec((1, two_im, D), lambda i, e: (e, 0, 0)),
            pl.BlockSpec((1, i_moe, D), lambda i, e: (e, 0, 0)),
            pl.BlockSpec((bt, D), lambda i, e: (i, 0)),
        ],
        out_specs=pl.BlockSpec((bt, D), lambda i, e: (i, 0)),
        out_shape=jax.ShapeDtypeStruct((T, D), jnp.float32),
    )(x2, tw, ti, Wgu, WdT, part)


def kernel(hidden_states, positions, w_ln1, Wqkv, b_qkv, Wo, w_ln2, W_gate,
           W_sg, W_gu_s, W_d_s, W_gu, W_d):
    f32 = jnp.float32
    bf16 = jnp.bfloat16
    T, D = hidden_states.shape
    n_h, n_hkv, hd = NUM_H, NUM_HKV, HEAD_DIM
    n_qkv = Wqkv.shape[0]
    E = W_gate.shape[0]
    i_sh = W_d_s.shape[1]
    bt = _bdiv(T, 256)

    qkv = _qkv_proj(hidden_states, w_ln1, Wqkv.T, b_qkv, bt, _bdiv(n_qkv, 512))
    q = qkv[:, : n_h * hd].reshape(T, n_h, hd).transpose(1, 0, 2)
    k = qkv[:, n_h * hd: (n_h + n_hkv) * hd].reshape(T, n_hkv, hd).transpose(1, 0, 2)
    v = qkv[:, (n_h + n_hkv) * hd:].reshape(T, n_hkv, hd).transpose(1, 0, 2)

    inv = 1.0 / (ROPE_BASE ** (jnp.arange(0, hd, 2, dtype=f32) / hd))
    ang = positions.astype(f32)[:, None] * inv[None, :]
    qk_r = _rope(jnp.concatenate([q, k], axis=0), jnp.cos(ang), jnp.sin(ang))

    attn = _attention(qk_r[:n_h], qk_r[n_h:], v.astype(bf16), _bdiv(T, 512))
    o_flat = attn.transpose(1, 0, 2).reshape(T, n_h * hd).astype(bf16)
    h = _oproj(o_flat, Wo.T.astype(bf16), hidden_states, bt, _bdiv(D, 512))

    wr = jnp.concatenate(
        [W_gate.T, W_sg.T, jnp.zeros((D, 128 - E - 1), f32)], axis=1)
    x2_bf, tw, ti, sgate = _router(h, w_ln2, wr, bt, E)

    act_s = _gu(x2_bf, W_gu_s[:i_sh].T.astype(bf16),
                W_gu_s[i_sh:].T.astype(bf16), bt, _bdiv(i_sh, 512))
    part = _down(act_s, W_d_s.T.astype(bf16), h, sgate, bt, _bdiv(D, 512))

    out = _moe(x2_bf, tw, ti, W_gu.astype(bf16),
               jnp.transpose(W_d, (0, 2, 1)).astype(bf16), part, bt)
    return out
